# baseline (speedup 1.0000x reference)
import functools

import jax
import jax.numpy as jnp
from jax import lax
from jax.experimental import pallas as pl
from jax.experimental.pallas import tpu as pltpu
from jax.experimental.pallas import tpu_sc as plsc

_NC = 2
_NS = 16
_NW = _NC * _NS

_SUB = 128
_CH = 256
_NBUF = 2


def _sc_gather_call(n_pad, d, k_chunks):
    n_sub = _CH // _SUB

    mesh = plsc.VectorSubcoreMesh(
        core_axis_name="c", subcore_axis_name="s",
        num_cores=_NC, num_subcores=_NS,
    )

    @functools.partial(
        pl.kernel,
        out_type=(
            jax.ShapeDtypeStruct((n_pad, d), jnp.float32),
            jax.ShapeDtypeStruct((n_pad, d), jnp.float32),
            jax.ShapeDtypeStruct((n_pad, d), jnp.float32),
        ),
        mesh=mesh,
        scratch_types=[
            pltpu.VMEM((_NBUF * _CH,), jnp.int32),
            pltpu.VMEM((_NBUF * _CH,), jnp.int32),
            pltpu.VMEM((_NBUF * _CH,), jnp.int32),
            pltpu.VMEM((_NBUF * _CH, d), jnp.float32),
            pltpu.VMEM((_NBUF * _CH, d), jnp.float32),
            pltpu.VMEM((_NBUF * _CH, d), jnp.float32),
            pltpu.SemaphoreType.DMA,
            pltpu.SemaphoreType.DMA,
            pltpu.SemaphoreType.DMA,
        ],
        compiler_params=pltpu.CompilerParams(use_tc_tiling_on_sc=False),
    )
    def sc_gather(atom_hbm, bond_hbm, iat_hbm, ibi_hbm, ibj_hbm,
                  out_at, out_bi, out_bj,
                  iat_v, ibi_v, ibj_v, rat_v, rbi_v, rbj_v,
                  sem_at, sem_bi, sem_bj):
        wid = lax.axis_index("s") * _NC + lax.axis_index("c")
        row_base = wid * (k_chunks * _CH)

        def fire(k):
            par = lax.rem(k, _NBUF)
            row0 = row_base + k * _CH
            i_off = par * _CH
            rows = pl.ds(row0, _CH)
            i_dst = pl.ds(i_off, _CH)
            pltpu.sync_copy(iat_hbm.at[rows], iat_v.at[i_dst])
            pltpu.sync_copy(ibi_hbm.at[rows], ibi_v.at[i_dst])
            pltpu.sync_copy(ibj_hbm.at[rows], ibj_v.at[i_dst])
            for j in range(n_sub):
                dst = pl.ds(par * _CH + j * _SUB, _SUB)
                isl = pl.ds(i_off + j * _SUB, _SUB)
                pltpu.async_copy(
                    atom_hbm.at[iat_v.at[isl]], rat_v.at[dst], sem_at)
                pltpu.async_copy(
                    bond_hbm.at[ibi_v.at[isl]], rbi_v.at[dst], sem_bi)
                pltpu.async_copy(
                    bond_hbm.at[ibj_v.at[isl]], rbj_v.at[dst], sem_bj)

        def drain_store(k):
            par = lax.rem(k, _NBUF)
            buf = pl.ds(par * _CH, _CH)
            row0 = row_base + k * _CH
            dst = out_at.at[pl.ds(row0, _CH)]
            dst_bi = out_bi.at[pl.ds(row0, _CH)]
            dst_bj = out_bj.at[pl.ds(row0, _CH)]
            pltpu.make_async_copy(dst, rat_v.at[buf], sem_at).wait()
            pltpu.make_async_copy(dst_bi, rbi_v.at[buf], sem_bi).wait()
            pltpu.make_async_copy(dst_bj, rbj_v.at[buf], sem_bj).wait()
            pltpu.sync_copy(rat_v.at[buf], dst)
            pltpu.sync_copy(rbi_v.at[buf], dst_bi)
            pltpu.sync_copy(rbj_v.at[buf], dst_bj)

        fire(0)

        def body(k, carry):
            @pl.when(k + 1 < k_chunks)
            def _():
                fire(k + 1)
            drain_store(k)
            return carry

        lax.fori_loop(0, k_chunks, body, 0)

    return sc_gather


def _tc_mlp_call(n, d, n_pad, blk):
    grid = (n // (2 * blk),)

    def body(bi_ref, bj_ref, ang_ref, at_ref, w_ref, b_ref, out_ref):
        w = w_ref[...]
        b = b_ref[...]
        ang = ang_ref[...]

        def half(h):
            sl = slice(h * d, (h + 1) * d)
            x = jnp.concatenate(
                [bi_ref[:, sl], bj_ref[:, sl], ang[:, sl], at_ref[:, sl]],
                axis=1)
            z = jnp.dot(x, w, preferred_element_type=jnp.float32) + b
            c = z[:, :d]
            g = z[:, d:]
            return c * jax.nn.sigmoid(c) * jax.nn.sigmoid(g) + ang[:, sl]

        out_ref[...] = jnp.concatenate([half(0), half(1)], axis=1)

    row_spec = pl.BlockSpec((blk, 2 * d), lambda i: (i, 0))
    return pl.pallas_call(
        body,
        grid=grid,
        in_specs=[
            row_spec,
            row_spec,
            row_spec,
            row_spec,
            pl.BlockSpec((4 * d, 2 * d), lambda i: (0, 0)),
            pl.BlockSpec((1, 2 * d), lambda i: (0, 0)),
        ],
        out_specs=row_spec,
        out_shape=jax.ShapeDtypeStruct((n // 2, 2 * d), jnp.float32),
    )


def kernel(atom_feas, bond_feas, angle_feas, bond_graph,
           W_core, b_core, W_gate, b_gate):
    n, d = angle_feas.shape
    per_w = _CH * _NW
    k_chunks = -(-n // per_w)
    n_pad = k_chunks * per_w
    bg = jnp.pad(bond_graph, ((0, n_pad - n), (0, 0))).reshape(n_pad * 3)

    g_at, g_bi, g_bj = _sc_gather_call(n_pad, d, k_chunks)(
        atom_feas, bond_feas, bg[0::3], bg[1::3], bg[2::3])

    w_cat = jnp.concatenate([W_core, W_gate], axis=1)
    b_cat = jnp.concatenate([b_core, b_gate]).reshape(1, 2 * d)
    ang2 = angle_feas.reshape(n // 2, 2 * d)

    out2 = _tc_mlp_call(n, d, n_pad, 1000)(
        g_bi.reshape(n_pad // 2, 2 * d),
        g_bj.reshape(n_pad // 2, 2 * d),
        ang2,
        g_at.reshape(n_pad // 2, 2 * d),
        w_cat, b_cat)
    return out2.reshape(n, d)


# speedup vs baseline: 1.3629x; 1.3629x over previous
import functools

import jax
import jax.numpy as jnp
from jax import lax
from jax.experimental import pallas as pl
from jax.experimental.pallas import tpu as pltpu
from jax.experimental.pallas import tpu_sc as plsc

_NC = 2
_NS = 16
_NW = _NC * _NS

_SUB = 128
_CH = 256
_NBUF = 2


def _sc_gather_call(n_pad, d, k_chunks):
    n_sub = _CH // _SUB

    mesh = plsc.VectorSubcoreMesh(
        core_axis_name="c", subcore_axis_name="s",
        num_cores=_NC, num_subcores=_NS,
    )

    @functools.partial(
        pl.kernel,
        out_type=(
            jax.ShapeDtypeStruct((n_pad, d), jnp.float32),
            jax.ShapeDtypeStruct((n_pad, d), jnp.float32),
            jax.ShapeDtypeStruct((n_pad, d), jnp.float32),
        ),
        mesh=mesh,
        scratch_types=[
            pltpu.VMEM((_NBUF * _CH,), jnp.int32),
            pltpu.VMEM((_NBUF * _CH,), jnp.int32),
            pltpu.VMEM((_NBUF * _CH,), jnp.int32),
            pltpu.VMEM((_NBUF * _CH, d), jnp.float32),
            pltpu.VMEM((_NBUF * _CH, d), jnp.float32),
            pltpu.VMEM((_NBUF * _CH, d), jnp.float32),
            pltpu.SemaphoreType.DMA,
            pltpu.SemaphoreType.DMA,
            pltpu.SemaphoreType.DMA,
        ],
        compiler_params=pltpu.CompilerParams(use_tc_tiling_on_sc=False),
    )
    def sc_gather(atom_hbm, bond_hbm, iat_hbm, ibi_hbm, ibj_hbm,
                  out_at, out_bi, out_bj,
                  iat_v, ibi_v, ibj_v, rat_v, rbi_v, rbj_v,
                  sem_at, sem_bi, sem_bj):
        wid = lax.axis_index("s") * _NC + lax.axis_index("c")
        row_base = wid * (k_chunks * _CH)

        def fire(k):
            par = lax.rem(k, _NBUF)
            row0 = row_base + k * _CH
            i_off = par * _CH
            rows = pl.ds(row0, _CH)
            i_dst = pl.ds(i_off, _CH)
            pltpu.sync_copy(iat_hbm.at[rows], iat_v.at[i_dst])
            pltpu.sync_copy(ibi_hbm.at[rows], ibi_v.at[i_dst])
            pltpu.sync_copy(ibj_hbm.at[rows], ibj_v.at[i_dst])
            for j in range(n_sub):
                dst = pl.ds(par * _CH + j * _SUB, _SUB)
                isl = pl.ds(i_off + j * _SUB, _SUB)
                pltpu.async_copy(
                    atom_hbm.at[iat_v.at[isl]], rat_v.at[dst], sem_at)
                pltpu.async_copy(
                    bond_hbm.at[ibi_v.at[isl]], rbi_v.at[dst], sem_bi)
                pltpu.async_copy(
                    bond_hbm.at[ibj_v.at[isl]], rbj_v.at[dst], sem_bj)

        def drain_store(k):
            par = lax.rem(k, _NBUF)
            buf = pl.ds(par * _CH, _CH)
            row0 = row_base + k * _CH
            dst = out_at.at[pl.ds(row0, _CH)]
            dst_bi = out_bi.at[pl.ds(row0, _CH)]
            dst_bj = out_bj.at[pl.ds(row0, _CH)]
            pltpu.make_async_copy(dst, rat_v.at[buf], sem_at).wait()
            pltpu.make_async_copy(dst_bi, rbi_v.at[buf], sem_bi).wait()
            pltpu.make_async_copy(dst_bj, rbj_v.at[buf], sem_bj).wait()
            pltpu.sync_copy(rat_v.at[buf], dst)
            pltpu.sync_copy(rbi_v.at[buf], dst_bi)
            pltpu.sync_copy(rbj_v.at[buf], dst_bj)

        fire(0)

        def body(k, carry):
            @pl.when(k + 1 < k_chunks)
            def _():
                fire(k + 1)
            drain_store(k)
            return carry

        lax.fori_loop(0, k_chunks, body, 0)

    return sc_gather


def _tc_mlp_call(n, d, n_pad, blk):
    grid = (n // (2 * blk),)

    def body(bi_ref, bj_ref, ang_ref, at_ref, w_ref, b_ref, out_ref):
        w = w_ref[...]
        b = b_ref[...]
        ang = ang_ref[...]

        def half(h):
            sl = slice(h * d, (h + 1) * d)
            x = jnp.concatenate(
                [bi_ref[:, sl], bj_ref[:, sl], ang[:, sl], at_ref[:, sl]],
                axis=1)
            z = jnp.dot(x, w, preferred_element_type=jnp.float32) + b
            c = z[:, :d]
            g = z[:, d:]
            return c * jax.nn.sigmoid(c) * jax.nn.sigmoid(g) + ang[:, sl]

        h0 = half(0)
        h1 = half(1)
        out_ref[...] = jnp.concatenate(
            [h0[:, None, :], h1[:, None, :]], axis=1).reshape(2 * blk, d)

    row_spec = pl.BlockSpec((blk, 2 * d), lambda i: (i, 0))
    return pl.pallas_call(
        body,
        grid=grid,
        in_specs=[
            row_spec,
            row_spec,
            row_spec,
            row_spec,
            pl.BlockSpec((4 * d, 2 * d), lambda i: (0, 0)),
            pl.BlockSpec((1, 2 * d), lambda i: (0, 0)),
        ],
        out_specs=pl.BlockSpec((2 * blk, d), lambda i: (i, 0)),
        out_shape=jax.ShapeDtypeStruct((n, d), jnp.float32),
    )


def kernel(atom_feas, bond_feas, angle_feas, bond_graph,
           W_core, b_core, W_gate, b_gate):
    n, d = angle_feas.shape
    per_w = _CH * _NW
    k_chunks = -(-n // per_w)
    n_pad = k_chunks * per_w
    bg = jnp.pad(bond_graph, ((0, n_pad - n), (0, 0)))

    g_at, g_bi, g_bj = _sc_gather_call(n_pad, d, k_chunks)(
        atom_feas, bond_feas, bg[:, 0], bg[:, 1], bg[:, 2])

    w_cat = jnp.concatenate([W_core, W_gate], axis=1)
    b_cat = jnp.concatenate([b_core, b_gate]).reshape(1, 2 * d)
    ang2 = angle_feas.reshape(n // 2, 2 * d)

    return _tc_mlp_call(n, d, n_pad, 1000)(
        g_bi.reshape(n_pad // 2, 2 * d),
        g_bj.reshape(n_pad // 2, 2 * d),
        ang2,
        g_at.reshape(n_pad // 2, 2 * d),
        w_cat, b_cat)


# speedup vs baseline: 1.3784x; 1.0114x over previous
import functools

import jax
import jax.numpy as jnp
from jax import lax
from jax.experimental import pallas as pl
from jax.experimental.pallas import tpu as pltpu
from jax.experimental.pallas import tpu_sc as plsc

_NC = 2
_NS = 16
_NW = _NC * _NS

_SUB = 128
_CH = 256
_NBUF = 2


def _sc_gather_call(n_pad, d, k_chunks):
    n_sub = _CH // _SUB

    mesh = plsc.VectorSubcoreMesh(
        core_axis_name="c", subcore_axis_name="s",
        num_cores=_NC, num_subcores=_NS,
    )

    @functools.partial(
        pl.kernel,
        out_type=(
            jax.ShapeDtypeStruct((n_pad, d), jnp.float32),
            jax.ShapeDtypeStruct((n_pad, d), jnp.float32),
            jax.ShapeDtypeStruct((n_pad, d), jnp.float32),
        ),
        mesh=mesh,
        scratch_types=[
            pltpu.VMEM((_NBUF * _CH,), jnp.int32),
            pltpu.VMEM((_NBUF * _CH,), jnp.int32),
            pltpu.VMEM((_NBUF * _CH,), jnp.int32),
            pltpu.VMEM((_NBUF * _CH, d), jnp.float32),
            pltpu.VMEM((_NBUF * _CH, d), jnp.float32),
            pltpu.VMEM((_NBUF * _CH, d), jnp.float32),
            pltpu.SemaphoreType.DMA,
            pltpu.SemaphoreType.DMA,
            pltpu.SemaphoreType.DMA,
        ],
        compiler_params=pltpu.CompilerParams(use_tc_tiling_on_sc=False),
    )
    def sc_gather(atom_hbm, bond_hbm, iat_hbm, ibi_hbm, ibj_hbm,
                  out_at, out_bi, out_bj,
                  iat_v, ibi_v, ibj_v, rat_v, rbi_v, rbj_v,
                  sem_at, sem_bi, sem_bj):
        wid = lax.axis_index("s") * _NC + lax.axis_index("c")
        row_base = wid * (k_chunks * _CH)

        def fire(k):
            par = lax.rem(k, _NBUF)
            row0 = row_base + k * _CH
            i_off = par * _CH
            rows = pl.ds(row0, _CH)
            i_dst = pl.ds(i_off, _CH)
            pltpu.sync_copy(iat_hbm.at[rows], iat_v.at[i_dst])
            pltpu.sync_copy(ibi_hbm.at[rows], ibi_v.at[i_dst])
            pltpu.sync_copy(ibj_hbm.at[rows], ibj_v.at[i_dst])
            for j in range(n_sub):
                dst = pl.ds(par * _CH + j * _SUB, _SUB)
                isl = pl.ds(i_off + j * _SUB, _SUB)
                pltpu.async_copy(
                    atom_hbm.at[iat_v.at[isl]], rat_v.at[dst], sem_at)
                pltpu.async_copy(
                    bond_hbm.at[ibi_v.at[isl]], rbi_v.at[dst], sem_bi)
                pltpu.async_copy(
                    bond_hbm.at[ibj_v.at[isl]], rbj_v.at[dst], sem_bj)

        def drain_store(k):
            par = lax.rem(k, _NBUF)
            buf = pl.ds(par * _CH, _CH)
            row0 = row_base + k * _CH
            dst = out_at.at[pl.ds(row0, _CH)]
            dst_bi = out_bi.at[pl.ds(row0, _CH)]
            dst_bj = out_bj.at[pl.ds(row0, _CH)]
            pltpu.make_async_copy(dst, rat_v.at[buf], sem_at).wait()
            pltpu.make_async_copy(dst_bi, rbi_v.at[buf], sem_bi).wait()
            pltpu.make_async_copy(dst_bj, rbj_v.at[buf], sem_bj).wait()
            pltpu.sync_copy(rat_v.at[buf], dst)
            pltpu.sync_copy(rbi_v.at[buf], dst_bi)
            pltpu.sync_copy(rbj_v.at[buf], dst_bj)

        fire(0)

        def body(k, carry):
            @pl.when(k + 1 < k_chunks)
            def _():
                fire(k + 1)
            drain_store(k)
            return carry

        lax.fori_loop(0, k_chunks, body, 0)

    return sc_gather


def _tc_mlp_call(n, d, blk, n_rows, blk_off, aliased):
    grid = (n_rows // (2 * blk),)

    def body(*refs):
        bi_ref, bj_ref, ang_ref, at_ref, w_ref, b_ref, out_ref = refs[-7:]
        w = w_ref[...]
        b = b_ref[...]
        ang = ang_ref[...]

        def half(h):
            sl = slice(h * d, (h + 1) * d)
            x = jnp.concatenate(
                [bi_ref[:, sl], bj_ref[:, sl], ang[:, sl], at_ref[:, sl]],
                axis=1)
            z = jnp.dot(x, w, preferred_element_type=jnp.float32) + b
            c = z[:, :d]
            g = z[:, d:]
            return c * jax.nn.sigmoid(c) * jax.nn.sigmoid(g) + ang[:, sl]

        h0 = half(0)
        h1 = half(1)
        out_ref[...] = jnp.concatenate(
            [h0[:, None, :], h1[:, None, :]], axis=1).reshape(2 * blk, d)

    row_spec = pl.BlockSpec((blk, 2 * d), lambda i: (i, 0))
    ang_spec = pl.BlockSpec((blk, 2 * d), lambda i: (i + blk_off, 0))
    in_specs = [
        row_spec,
        row_spec,
        ang_spec,
        row_spec,
        pl.BlockSpec((4 * d, 2 * d), lambda i: (0, 0)),
        pl.BlockSpec((1, 2 * d), lambda i: (0, 0)),
    ]
    aliases = {}
    if aliased:
        in_specs = [pl.BlockSpec(memory_space=pltpu.MemorySpace.HBM)] + in_specs
        aliases = {0: 0}
    return pl.pallas_call(
        body,
        grid=grid,
        in_specs=in_specs,
        out_specs=pl.BlockSpec((2 * blk, d), lambda i: (i + blk_off, 0)),
        out_shape=jax.ShapeDtypeStruct((n, d), jnp.float32),
        input_output_aliases=aliases,
    )


def kernel(atom_feas, bond_feas, angle_feas, bond_graph,
           W_core, b_core, W_gate, b_gate):
    n, d = angle_feas.shape
    per_w = _CH * _NW
    k_chunks = -(-n // per_w)
    n_pad = k_chunks * per_w
    bg = jnp.pad(bond_graph, ((0, n_pad - n), (0, 0)))

    w_cat = jnp.concatenate([W_core, W_gate], axis=1)
    b_cat = jnp.concatenate([b_core, b_gate]).reshape(1, 2 * d)
    ang2 = angle_feas.reshape(n // 2, 2 * d)

    blk = 800
    k0 = k_chunks // 2 + 1
    k1 = k_chunks - k0
    n0 = k0 * per_w
    n1_valid = n - n0
    assert n0 % (2 * blk) == 0 and n1_valid % (2 * blk) == 0

    def stage_gather(k_cnt, lo, hi):
        return _sc_gather_call(k_cnt * per_w, d, k_cnt)(
            atom_feas, bond_feas,
            bg[lo:hi, 0], bg[lo:hi, 1], bg[lo:hi, 2])

    g_at0, g_bi0, g_bj0 = stage_gather(k0, 0, n0)
    g_at1, g_bi1, g_bj1 = stage_gather(k1, n0, n_pad)

    def pair(x):
        return x.reshape(x.shape[0] // 2, 2 * d)

    out0 = _tc_mlp_call(n, d, blk, n0, 0, False)(
        pair(g_bi0), pair(g_bj0), ang2, pair(g_at0), w_cat, b_cat)
    out = _tc_mlp_call(n, d, blk, n1_valid, n0 // (2 * blk), True)(
        out0, pair(g_bi1), pair(g_bj1), ang2, pair(g_at1), w_cat, b_cat)
    return out
